# baseline (speedup 1.0000x reference)
import jax
import jax.numpy as jnp
from jax import lax
from jax.experimental import pallas as pl
from jax.experimental.pallas import tpu as pltpu
from jax.experimental.pallas import tpu_sc as plsc

NUM_NUMERICAL = 13
N_CAT = 26
CARD = 100000
D_TOKEN = 32
BATCH = 16384
N_TOK = 1 + NUM_NUMERICAL + N_CAT
NROW = N_CAT * CARD

NW = 32
R = BATCH // NW
C = 32
G = R // C
L = 16
DB = D_TOKEN // 8
BB = BATCH // 128
TROW = N_TOK * DB

NBLK = (NROW + 127) // 128
LASTB = NBLK - 1
TPB = (NBLK + NW - 1) // NW
NCH = LASTB // 2
TPC = (NCH + NW - 1) // NW
NRING = 4


def _conv_body(tT_hbm, out_hbm,
               v_0, v_1, v_2, v_3, ob_0, ob_1, ob_2, ob_3, v64,
               si_0, si_1, si_2, si_3, so_0, so_1, so_2, so_3):
    wid = lax.axis_index("s") * 2 + lax.axis_index("c")
    lane = lax.iota(jnp.int32, L)
    c_lo = lane
    c_hi = lane + L
    kcol = [jnp.full((L,), kk, jnp.int32) for kk in range(4)]
    vs = [v_0, v_1, v_2, v_3]
    obs = [ob_0, ob_1, ob_2, ob_3]
    sis = [si_0, si_1, si_2, si_3]
    sos = [so_0, so_1, so_2, so_3]

    def chk(t):
        return jnp.minimum(t * NW + wid, NCH - 1)

    def start_in(ch, v, sem):
        return pltpu.async_copy(tT_hbm.at[:, pl.ds(ch * 256, 256)], v, sem)

    def drain_in(v, sem):
        pltpu.make_async_copy(tT_hbm.at[:, pl.ds(0, 256)], v, sem).wait()

    def drain_out(ob, sem):
        pltpu.make_async_copy(out_hbm.at[pl.ds(0, 64), :], ob, sem).wait()

    def transpose_chk(v, ob):
        def jq_body(jq, c):
            for b4 in range(2):
                base = 128 * b4 + 4 * jq
                vals = [plsc.load_gather(
                    v, [(c_hi if (k & 1) else c_lo), kcol[k // 2] + base])
                    for k in range(8)]
                for k in range(8):
                    ob[32 * b4 + jq, pl.ds(16 * k, L)] = vals[k]
            return c
        lax.fori_loop(0, 32, jq_body, 0)

    for b in range(NRING):
        start_in(chk(b), vs[b], sis[b])

    def ring(tt, carry):
        t0 = tt * NRING
        for b in range(NRING):
            t = t0 + b

            @pl.when(tt > 0)
            def _(b=b):
                drain_out(obs[b], sos[b])
            drain_in(vs[b], sis[b])
            transpose_chk(vs[b], obs[b])
            start_in(chk(t + NRING), vs[b], sis[b])
            pltpu.async_copy(obs[b],
                             out_hbm.at[pl.ds(chk(t) * 64, 64), :], sos[b])
        return carry
    lax.fori_loop(0, TPC // NRING, ring, 0)
    for b in range(NRING):
        t = (TPC // NRING) * NRING + b
        drain_out(obs[b], sos[b])
        drain_in(vs[b], sis[b])
        transpose_chk(vs[b], obs[b])
        pltpu.async_copy(obs[b],
                         out_hbm.at[pl.ds(chk(t) * 64, 64), :], sos[b])
    for b in range(NRING):
        drain_out(obs[b], sos[b])

    @pl.when(wid == NW - 1)
    def _tail():
        ncol = NROW - LASTB * 128
        pltpu.sync_copy(tT_hbm.at[:, pl.ds(LASTB * 128, ncol)], v64)
        def jq_body(jq, c):
            for k in range(8):
                src_c = c_hi if (k & 1) else c_lo
                col = jnp.full((L,), 4 * jq + k // 2, jnp.int32)
                ob_0[jq, pl.ds(16 * k, L)] = plsc.load_gather(v64,
                                                              [src_c, col])
            return c
        lax.fori_loop(0, ncol // 4, jq_body, 0)
        pltpu.sync_copy(ob_0.at[pl.ds(0, ncol // 4), :],
                        out_hbm.at[pl.ds(LASTB * 32, ncol // 4), :])


def _sc_body(xnum_hbm, idxT_hbm, w_hbm, nb_hbm, table_hbm, cb_hbm, cls_hbm,
             out_hbm,
             idx_v, rows_v, stage_v, xnum_v, w_v, nb_v, cb_v, cls_v,
             sem_g, sem_o):
    wid = lax.axis_index("s") * 2 + lax.axis_index("c")
    base = wid * R

    pltpu.sync_copy(xnum_hbm.at[pl.ds(base, R)], xnum_v)
    pltpu.sync_copy(w_hbm, w_v)
    pltpu.sync_copy(nb_hbm, nb_v)
    pltpu.sync_copy(cb_hbm, cb_v)
    pltpu.sync_copy(cls_hbm, cls_v)

    cls0 = cls_v[pl.ds(0, L)]
    cls1 = cls_v[pl.ds(L, L)]

    lane = lax.iota(jnp.int32, L)
    db_lo = lane // 8
    db_hi = db_lo + 2
    d_in = lane % 8

    def scat(t4, r, v0, v1):
        rv = jnp.full((L,), r, jnp.int32)
        plsc.store_scatter(stage_v, [t4 + db_lo, d_in, rv], v0)
        plsc.store_scatter(stage_v, [t4 + db_hi, d_in, rv], v1)

    def chunk(g, carry):
        row0 = base + g * C

        pltpu.sync_copy(idxT_hbm.at[:, pl.ds(row0, C)], idx_v)

        gathers = []
        for f in range(N_CAT):
            gathers.append(
                pltpu.async_copy(table_hbm.at[idx_v.at[f]], rows_v.at[f],
                                 sem_g))

        def cls_body(r, c):
            scat(0, r, cls0, cls1)
            return c
        lax.fori_loop(0, C, cls_body, 0)

        for i in range(NUM_NUMERICAL):
            w0 = w_v[i, pl.ds(0, L)]
            w1 = w_v[i, pl.ds(L, L)]
            b0 = nb_v[i, pl.ds(0, L)]
            b1 = nb_v[i, pl.ds(L, L)]

            def num_body(r, c, w0=w0, w1=w1, b0=b0, b1=b1, i=i):
                xs = xnum_v[g * C + r, :][i]
                scat((1 + i) * 4, r, xs * w0 + b0, xs * w1 + b1)
                return c
            lax.fori_loop(0, C, num_body, 0)

        for cp in gathers:
            cp.wait()

        def cat_body(f, c):
            c0 = cb_v[f, pl.ds(0, L)]
            c1 = cb_v[f, pl.ds(L, L)]
            t4 = (1 + NUM_NUMERICAL + f) * 4

            def row_body(r, cc):
                v0 = rows_v[f, r, pl.ds(0, L)] + c0
                v1 = rows_v[f, r, pl.ds(L, L)] + c1
                scat(t4, r, v0, v1)
                return cc
            lax.fori_loop(0, C, row_body, 0)
            return c
        lax.fori_loop(0, N_CAT, cat_body, 0)

        bb = wid * (R // 128) + g // (128 // C)
        h = g % (128 // C)
        cp = pltpu.async_copy(
            stage_v, out_hbm.at[:, bb, :, pl.ds(h * C, C)], sem_o)
        cp.wait()
        return carry

    lax.fori_loop(0, G, chunk, 0)


@jax.jit
def kernel(x_num, x_cat, num_weight, num_bias, table, cat_bias, cls):
    offsets = (jnp.arange(N_CAT, dtype=jnp.int32) * CARD)[:, None]
    idxT = x_cat.astype(jnp.int32).T + offsets
    x_num_p = jnp.pad(x_num, ((0, 0), (0, L - NUM_NUMERICAL)))

    mesh = plsc.VectorSubcoreMesh(core_axis_name="c", subcore_axis_name="s")

    conv = pl.kernel(
        _conv_body,
        out_type=jax.ShapeDtypeStruct((NROW // 4, 128), jnp.float32),
        mesh=mesh,
        compiler_params=pltpu.CompilerParams(use_tc_tiling_on_sc=True,
                                             needs_layout_passes=False),
        scratch_types=(
            [pltpu.VMEM((D_TOKEN, 256), jnp.float32) for _ in range(NRING)]
            + [pltpu.VMEM((64, 128), jnp.float32) for _ in range(NRING)]
            + [pltpu.VMEM((D_TOKEN, 64), jnp.float32)]
            + [pltpu.SemaphoreType.DMA for _ in range(2 * NRING)]
        ),
    )
    tbl_c = conv(table.T)
    tbl_rm = tbl_c.reshape(NROW, D_TOKEN)

    call = pl.kernel(
        _sc_body,
        out_type=jax.ShapeDtypeStruct((TROW, BB, 8, 128), jnp.float32),
        mesh=mesh,
        compiler_params=pltpu.CompilerParams(use_tc_tiling_on_sc=False,
                                             needs_layout_passes=False),
        scratch_types=[
            pltpu.VMEM((N_CAT, C), jnp.int32),
            pltpu.VMEM((N_CAT, C, D_TOKEN), jnp.float32),
            pltpu.VMEM((TROW, 8, C), jnp.float32),
            pltpu.VMEM((R, L), jnp.float32),
            pltpu.VMEM((NUM_NUMERICAL, D_TOKEN), jnp.float32),
            pltpu.VMEM((NUM_NUMERICAL, D_TOKEN), jnp.float32),
            pltpu.VMEM((N_CAT, D_TOKEN), jnp.float32),
            pltpu.VMEM((D_TOKEN,), jnp.float32),
            pltpu.SemaphoreType.DMA,
            pltpu.SemaphoreType.DMA,
        ],
    )
    out5 = call(x_num_p, idxT, num_weight, num_bias, tbl_rm, cat_bias, cls)
    out5 = out5.reshape(N_TOK, DB, BB, 8, 128)
    return out5.transpose(2, 4, 0, 1, 3).reshape(BATCH, N_TOK, D_TOKEN)


# speedup vs baseline: 1.0393x; 1.0393x over previous
import jax
import jax.numpy as jnp
from jax import lax
from jax.experimental import pallas as pl
from jax.experimental.pallas import tpu as pltpu
from jax.experimental.pallas import tpu_sc as plsc

NUM_NUMERICAL = 13
N_CAT = 26
CARD = 100000
D_TOKEN = 32
BATCH = 16384
N_TOK = 1 + NUM_NUMERICAL + N_CAT
NROW = N_CAT * CARD

NW = 32
R = BATCH // NW
C = 32
G = R // C
L = 16
DB = D_TOKEN // 8
BB = BATCH // 128
TROW = N_TOK * DB

NBLK = (NROW + 127) // 128
LASTB = NBLK - 1
TPB = (NBLK + NW - 1) // NW
NCH = LASTB // 2
TPC = (NCH + NW - 1) // NW
NRING = 4


def _conv_body(tT_hbm, out_hbm,
               v_0, v_1, v_2, v_3, ob_0, ob_1, ob_2, ob_3, v64,
               si_0, si_1, si_2, si_3, so_0, so_1, so_2, so_3):
    wid = lax.axis_index("s") * 2 + lax.axis_index("c")
    lane = lax.iota(jnp.int32, L)
    c_lo = lane
    c_hi = lane + L
    kcol = [jnp.full((L,), kk, jnp.int32) for kk in range(4)]
    vs = [v_0, v_1, v_2, v_3]
    obs = [ob_0, ob_1, ob_2, ob_3]
    sis = [si_0, si_1, si_2, si_3]
    sos = [so_0, so_1, so_2, so_3]

    def chk(t):
        return jnp.minimum(t * NW + wid, NCH - 1)

    def start_in(ch, v, sem):
        return pltpu.async_copy(tT_hbm.at[:, pl.ds(ch * 256, 256)], v, sem)

    def drain_in(v, sem):
        pltpu.make_async_copy(tT_hbm.at[:, pl.ds(0, 256)], v, sem).wait()

    def drain_out(ob, sem):
        pltpu.make_async_copy(out_hbm.at[pl.ds(0, 64), :], ob, sem).wait()

    def transpose_chk(v, ob):
        def jq_body(jq, c):
            for b4 in range(2):
                base = 128 * b4 + 4 * jq
                vals = [plsc.load_gather(
                    v, [(c_hi if (k & 1) else c_lo), kcol[k // 2] + base])
                    for k in range(8)]
                for k in range(8):
                    ob[32 * b4 + jq, pl.ds(16 * k, L)] = vals[k]
            return c
        lax.fori_loop(0, 32, jq_body, 0)

    for b in range(NRING):
        start_in(chk(b), vs[b], sis[b])

    def ring(tt, carry):
        t0 = tt * NRING
        for b in range(NRING):
            t = t0 + b

            @pl.when(tt > 0)
            def _(b=b):
                drain_out(obs[b], sos[b])
            drain_in(vs[b], sis[b])
            transpose_chk(vs[b], obs[b])
            start_in(chk(t + NRING), vs[b], sis[b])
            pltpu.async_copy(obs[b],
                             out_hbm.at[pl.ds(chk(t) * 64, 64), :], sos[b])
        return carry
    lax.fori_loop(0, TPC // NRING, ring, 0)
    for b in range(NRING):
        t = (TPC // NRING) * NRING + b
        drain_out(obs[b], sos[b])
        drain_in(vs[b], sis[b])
        transpose_chk(vs[b], obs[b])
        pltpu.async_copy(obs[b],
                         out_hbm.at[pl.ds(chk(t) * 64, 64), :], sos[b])
    for b in range(NRING):
        drain_out(obs[b], sos[b])

    @pl.when(wid == NW - 1)
    def _tail():
        ncol = NROW - LASTB * 128
        pltpu.sync_copy(tT_hbm.at[:, pl.ds(LASTB * 128, ncol)], v64)
        def jq_body(jq, c):
            for k in range(8):
                src_c = c_hi if (k & 1) else c_lo
                col = jnp.full((L,), 4 * jq + k // 2, jnp.int32)
                ob_0[jq, pl.ds(16 * k, L)] = plsc.load_gather(v64,
                                                              [src_c, col])
            return c
        lax.fori_loop(0, ncol // 4, jq_body, 0)
        pltpu.sync_copy(ob_0.at[pl.ds(0, ncol // 4), :],
                        out_hbm.at[pl.ds(LASTB * 32, ncol // 4), :])


def _sc_body(xnum_hbm, idxT_hbm, w_hbm, nb_hbm, table_hbm, cb_hbm, cls_hbm,
             out_hbm,
             idx_v, rows_v, stage_v, xnum_v, w_v, nb_v, cb_v, cls_v,
             sem_g, sem_o):
    wid = lax.axis_index("s") * 2 + lax.axis_index("c")
    base = wid * R

    pltpu.sync_copy(xnum_hbm.at[pl.ds(base, R)], xnum_v)
    pltpu.sync_copy(w_hbm, w_v)
    pltpu.sync_copy(nb_hbm, nb_v)
    pltpu.sync_copy(cb_hbm, cb_v)
    pltpu.sync_copy(cls_hbm, cls_v)

    cls0 = cls_v[pl.ds(0, L)]
    cls1 = cls_v[pl.ds(L, L)]

    lane = lax.iota(jnp.int32, L)
    db_lo = lane // 8
    db_hi = db_lo + 2
    d_in = lane % 8

    def scat(t4, r, v0, v1):
        rv = jnp.full((L,), r, jnp.int32)
        plsc.store_scatter(stage_v, [t4 + db_lo, d_in, rv], v0)
        plsc.store_scatter(stage_v, [t4 + db_hi, d_in, rv], v1)

    def chunk(g, carry):
        row0 = base + g * C

        pltpu.sync_copy(idxT_hbm.at[:, pl.ds(row0, C)], idx_v)

        gathers = []
        for f in range(N_CAT):
            gathers.append(
                pltpu.async_copy(table_hbm.at[idx_v.at[f]], rows_v.at[f],
                                 sem_g))

        def cls_body(r4, c):
            for u in range(4):
                scat(0, r4 * 4 + u, cls0, cls1)
            return c
        lax.fori_loop(0, C // 4, cls_body, 0)

        for i in range(NUM_NUMERICAL):
            w0 = w_v[i, pl.ds(0, L)]
            w1 = w_v[i, pl.ds(L, L)]
            b0 = nb_v[i, pl.ds(0, L)]
            b1 = nb_v[i, pl.ds(L, L)]

            def num_body(r4, c, w0=w0, w1=w1, b0=b0, b1=b1, i=i):
                vals = []
                for u in range(4):
                    r = r4 * 4 + u
                    xs = xnum_v[g * C + r, :][i]
                    vals.append((r, xs * w0 + b0, xs * w1 + b1))
                for r, a0, a1 in vals:
                    scat((1 + i) * 4, r, a0, a1)
                return c
            lax.fori_loop(0, C // 4, num_body, 0)

        for cp in gathers:
            cp.wait()

        def cat_body(f, c):
            c0 = cb_v[f, pl.ds(0, L)]
            c1 = cb_v[f, pl.ds(L, L)]
            t4 = (1 + NUM_NUMERICAL + f) * 4

            def row_body(r4, cc):
                vals = []
                for u in range(4):
                    r = r4 * 4 + u
                    vals.append((r,
                                 rows_v[f, r, pl.ds(0, L)] + c0,
                                 rows_v[f, r, pl.ds(L, L)] + c1))
                for r, a0, a1 in vals:
                    scat(t4, r, a0, a1)
                return cc
            lax.fori_loop(0, C // 4, row_body, 0)
            return c
        lax.fori_loop(0, N_CAT, cat_body, 0)

        bb = wid * (R // 128) + g // (128 // C)
        h = g % (128 // C)
        cp = pltpu.async_copy(
            stage_v, out_hbm.at[:, bb, :, pl.ds(h * C, C)], sem_o)
        cp.wait()
        return carry

    lax.fori_loop(0, G, chunk, 0)


@jax.jit
def kernel(x_num, x_cat, num_weight, num_bias, table, cat_bias, cls):
    offsets = (jnp.arange(N_CAT, dtype=jnp.int32) * CARD)[:, None]
    idxT = x_cat.astype(jnp.int32).T + offsets
    x_num_p = jnp.pad(x_num, ((0, 0), (0, L - NUM_NUMERICAL)))

    mesh = plsc.VectorSubcoreMesh(core_axis_name="c", subcore_axis_name="s")

    conv = pl.kernel(
        _conv_body,
        out_type=jax.ShapeDtypeStruct((NROW // 4, 128), jnp.float32),
        mesh=mesh,
        compiler_params=pltpu.CompilerParams(use_tc_tiling_on_sc=True,
                                             needs_layout_passes=False),
        scratch_types=(
            [pltpu.VMEM((D_TOKEN, 256), jnp.float32) for _ in range(NRING)]
            + [pltpu.VMEM((64, 128), jnp.float32) for _ in range(NRING)]
            + [pltpu.VMEM((D_TOKEN, 64), jnp.float32)]
            + [pltpu.SemaphoreType.DMA for _ in range(2 * NRING)]
        ),
    )
    tbl_c = conv(table.T)
    tbl_rm = tbl_c.reshape(NROW, D_TOKEN)

    call = pl.kernel(
        _sc_body,
        out_type=jax.ShapeDtypeStruct((TROW, BB, 8, 128), jnp.float32),
        mesh=mesh,
        compiler_params=pltpu.CompilerParams(use_tc_tiling_on_sc=False,
                                             needs_layout_passes=False),
        scratch_types=[
            pltpu.VMEM((N_CAT, C), jnp.int32),
            pltpu.VMEM((N_CAT, C, D_TOKEN), jnp.float32),
            pltpu.VMEM((TROW, 8, C), jnp.float32),
            pltpu.VMEM((R, L), jnp.float32),
            pltpu.VMEM((NUM_NUMERICAL, D_TOKEN), jnp.float32),
            pltpu.VMEM((NUM_NUMERICAL, D_TOKEN), jnp.float32),
            pltpu.VMEM((N_CAT, D_TOKEN), jnp.float32),
            pltpu.VMEM((D_TOKEN,), jnp.float32),
            pltpu.SemaphoreType.DMA,
            pltpu.SemaphoreType.DMA,
        ],
    )
    out5 = call(x_num_p, idxT, num_weight, num_bias, tbl_rm, cat_bias, cls)
    out5 = out5.reshape(N_TOK, DB, BB, 8, 128)
    return out5.transpose(2, 4, 0, 1, 3).reshape(BATCH, N_TOK, D_TOKEN)
